# baseline (speedup 1.0000x reference)
import jax
import jax.numpy as jnp
from jax.experimental import pallas as pl


def _gnn_kernel(unt_ref, adj_ref,
                wemb_ref, bemb_ref,
                wroot0_ref, wrelc0_ref, b0_ref,
                wroot1_ref, wrelc1_ref, b1_ref,
                wpol_ref, bpol_ref, wbase_ref, bbase_ref,
                logits_ref, base_ref, act_ref):
    f32 = jnp.float32
    A = wpol_ref.shape[1]
    D = wroot0_ref.shape[0]
    RD, N = wrelc0_ref.shape[1], unt_ref.shape[1]
    R = RD // D
    NR = N * R

    adjf = adj_ref[0].astype(f32)
    deg = jnp.sum(adjf, axis=0, keepdims=True)
    adjh = adjf * (1.0 / jnp.maximum(deg, 1.0))

    xT = jax.lax.dot_general(wemb_ref[...], unt_ref[0],
                             (((0,), (1,)), ((), ())),
                             precision=jax.lax.Precision.HIGHEST,
                             preferred_element_type=f32) + bemb_ref[...]

    prow = jax.lax.broadcasted_iota(jnp.int32, (RD, NR), 0) // D
    ccol = jax.lax.broadcasted_iota(jnp.int32, (RD, NR), 1) % R
    mask = (prow == ccol).astype(f32)

    def rgcn(xT, wroot, wrelc, bias):
        hall = jax.lax.dot_general(wrelc, xT, (((0,), (0,)), ((), ())),
                                   precision=jax.lax.Precision.HIGHEST,
                                   preferred_element_type=f32)
        g = jax.lax.dot_general(hall, adjh, (((1,), (0,)), ((), ())),
                                precision=jax.lax.Precision.HIGHEST,
                                preferred_element_type=f32)
        gm = g * mask
        s1g = gm[0 * D:1 * D] + gm[1 * D:2 * D] + gm[2 * D:3 * D] + gm[3 * D:4 * D]
        aggT = jnp.sum(s1g.reshape(D, N, R), axis=2)
        rootT = jax.lax.dot_general(wroot, xT, (((0,), (0,)), ((), ())),
                                    precision=jax.lax.Precision.HIGHEST,
                                    preferred_element_type=f32)
        return jax.nn.relu(rootT + bias + aggT)

    xT = rgcn(xT, wroot0_ref[...], wrelc0_ref[...], b0_ref[...])
    xT = rgcn(xT, wroot1_ref[...], wrelc1_ref[...], b1_ref[...])

    pooled = jnp.max(xT, axis=1, keepdims=True)
    logits = jax.lax.dot_general(pooled, wpol_ref[...], (((0,), (0,)), ((), ())),
                                 preferred_element_type=f32) + bpol_ref[...]
    base = jax.lax.dot_general(pooled, wbase_ref[...], (((0,), (0,)), ((), ())),
                               preferred_element_type=f32) + bbase_ref[...]

    logits_ref[0] = logits
    base_ref[0] = base
    m = jnp.max(logits, axis=1, keepdims=True)
    iota = jax.lax.broadcasted_iota(jnp.int32, logits.shape, 1)
    act_ref[0] = jnp.min(jnp.where(logits == m, iota, A), axis=1, keepdims=True)


def kernel(unary_tensor, binary_tensor, W_emb, b_emb, Wroot0, Wrel0, b0,
           Wroot1, Wrel1, b1, W_pol, b_pol, W_base, b_base):
    Tt, Bb, N, F = unary_tensor.shape
    R = binary_tensor.shape[-1]
    D = W_emb.shape[1]
    A = W_pol.shape[1]
    BT = Tt * Bb
    NR = N * R
    RD = R * D
    f32 = jnp.float32

    unt = unary_tensor.reshape(BT, N, F)
    adj = binary_tensor.reshape(BT, N, NR)

    wrelc0 = Wrel0.transpose(1, 0, 2).reshape(D, RD)
    wrelc1 = Wrel1.transpose(1, 0, 2).reshape(D, RD)

    full = lambda *shape: pl.BlockSpec(shape, lambda b: (0,) * len(shape))
    in_specs = [
        pl.BlockSpec((1, N, F), lambda b: (b, 0, 0)),
        pl.BlockSpec((1, N, NR), lambda b: (b, 0, 0)),
        full(F, D), full(D, 1),
        full(D, D), full(D, RD), full(D, 1),
        full(D, D), full(D, RD), full(D, 1),
        full(D, A), full(1, A), full(D, 1), full(1, 1),
    ]
    out_specs = [
        pl.BlockSpec((1, 1, A), lambda b: (b, 0, 0)),
        pl.BlockSpec((1, 1, 1), lambda b: (b, 0, 0)),
        pl.BlockSpec((1, 1, 1), lambda b: (b, 0, 0)),
    ]
    logits, base, act = pl.pallas_call(
        _gnn_kernel,
        grid=(BT,),
        in_specs=in_specs,
        out_specs=out_specs,
        out_shape=[
            jax.ShapeDtypeStruct((BT, 1, A), f32),
            jax.ShapeDtypeStruct((BT, 1, 1), f32),
            jax.ShapeDtypeStruct((BT, 1, 1), jnp.int32),
        ],
    )(unt, adj,
      W_emb, b_emb.reshape(D, 1),
      Wroot0, wrelc0, b0.reshape(D, 1),
      Wroot1, wrelc1, b1.reshape(D, 1),
      W_pol, b_pol.reshape(1, A), W_base, b_base.reshape(1, 1))

    return (logits.reshape(Tt, Bb, A),
            base.reshape(Tt, Bb),
            act.reshape(Tt, Bb))


# speedup vs baseline: 1.6565x; 1.6565x over previous
import jax
import jax.numpy as jnp
from jax.experimental import pallas as pl


def _gnn_kernel(unt_ref, adj_ref,
                wemb_ref, bemb_ref,
                wroot0_ref, wrel0_ref, b0_ref,
                wroot1_ref, wrel1_ref, b1_ref,
                wpol_ref, bpol_ref, wbase_ref, bbase_ref,
                logits_ref, base_ref, act_ref):
    f32 = jnp.float32
    HI = jax.lax.Precision.HIGHEST
    A = wpol_ref.shape[1]
    D = wroot0_ref.shape[0]
    R = wrel0_ref.shape[0]
    N = unt_ref.shape[1]

    adjf = adj_ref[0].astype(f32)
    deg = jnp.sum(adjf, axis=0, keepdims=True)
    adjh = adjf * (1.0 / jnp.maximum(deg, 1.0))

    xT = jax.lax.dot_general(wemb_ref[...], unt_ref[0],
                             (((0,), (1,)), ((), ())),
                             precision=HI,
                             preferred_element_type=f32) + bemb_ref[...]

    def rgcn(xT, wroot, wrel, bias):
        aggT = None
        for r in range(R):
            hT = jax.lax.dot_general(wrel[r], xT, (((0,), (0,)), ((), ())),
                                     precision=HI,
                                     preferred_element_type=f32)
            t = jax.lax.dot_general(hT, adjh[:, r * N:(r + 1) * N],
                                    (((1,), (0,)), ((), ())),
                                    precision=HI,
                                    preferred_element_type=f32)
            aggT = t if aggT is None else aggT + t
        rootT = jax.lax.dot_general(wroot, xT, (((0,), (0,)), ((), ())),
                                    precision=HI,
                                    preferred_element_type=f32)
        return jax.nn.relu(rootT + bias + aggT)

    xT = rgcn(xT, wroot0_ref[...], wrel0_ref[...], b0_ref[...])
    xT = rgcn(xT, wroot1_ref[...], wrel1_ref[...], b1_ref[...])

    pooled = jnp.max(xT, axis=1, keepdims=True)
    logits = jax.lax.dot_general(pooled, wpol_ref[...], (((0,), (0,)), ((), ())),
                                 precision=HI,
                                 preferred_element_type=f32) + bpol_ref[...]
    base = jax.lax.dot_general(pooled, wbase_ref[...], (((0,), (0,)), ((), ())),
                               precision=HI,
                               preferred_element_type=f32) + bbase_ref[...]

    logits_ref[0] = logits
    base_ref[0] = base
    m = jnp.max(logits, axis=1, keepdims=True)
    iota = jax.lax.broadcasted_iota(jnp.int32, logits.shape, 1)
    act_ref[0] = jnp.min(jnp.where(logits == m, iota, A), axis=1, keepdims=True)


def kernel(unary_tensor, binary_tensor, W_emb, b_emb, Wroot0, Wrel0, b0,
           Wroot1, Wrel1, b1, W_pol, b_pol, W_base, b_base):
    Tt, Bb, N, F = unary_tensor.shape
    R = binary_tensor.shape[-1]
    D = W_emb.shape[1]
    A = W_pol.shape[1]
    BT = Tt * Bb
    f32 = jnp.float32

    unt = unary_tensor.reshape(BT, N, F)
    adjt = binary_tensor.transpose(0, 1, 2, 4, 3).reshape(BT, N, R * N)

    full = lambda *shape: pl.BlockSpec(shape, lambda b: (0,) * len(shape))
    in_specs = [
        pl.BlockSpec((1, N, F), lambda b: (b, 0, 0)),
        pl.BlockSpec((1, N, R * N), lambda b: (b, 0, 0)),
        full(F, D), full(D, 1),
        full(D, D), full(R, D, D), full(D, 1),
        full(D, D), full(R, D, D), full(D, 1),
        full(D, A), full(1, A), full(D, 1), full(1, 1),
    ]
    out_specs = [
        pl.BlockSpec((1, 1, A), lambda b: (b, 0, 0)),
        pl.BlockSpec((1, 1, 1), lambda b: (b, 0, 0)),
        pl.BlockSpec((1, 1, 1), lambda b: (b, 0, 0)),
    ]
    logits, base, act = pl.pallas_call(
        _gnn_kernel,
        grid=(BT,),
        in_specs=in_specs,
        out_specs=out_specs,
        out_shape=[
            jax.ShapeDtypeStruct((BT, 1, A), f32),
            jax.ShapeDtypeStruct((BT, 1, 1), f32),
            jax.ShapeDtypeStruct((BT, 1, 1), jnp.int32),
        ],
    )(unt, adjt,
      W_emb, b_emb.reshape(D, 1),
      Wroot0, Wrel0, b0.reshape(D, 1),
      Wroot1, Wrel1, b1.reshape(D, 1),
      W_pol, b_pol.reshape(1, A), W_base, b_base.reshape(1, 1))

    return (logits.reshape(Tt, Bb, A),
            base.reshape(Tt, Bb),
            act.reshape(Tt, Bb))
